# baseline (speedup 1.0000x reference)
import functools

import jax
import jax.numpy as jnp
from jax import lax
from jax.experimental import pallas as pl
from jax.experimental.pallas import tpu as pltpu
from jax.experimental.pallas import tpu_sc as plsc

N = 10000
NP = 10240
F_IN = 128
F_HID = 128
F_OUT = 64
E = 320000
NC = 2
NS = 16
CH = 128
K = -(-E // (NC * NS * CH))
EP = NC * NS * CH * K
K0 = 51
K1 = 2 * K - K0
KMAX = max(K0, K1)
RPT = NP // NS
BR = NP // 8

_mesh = plsc.VectorSubcoreMesh(core_axis_name="c", subcore_axis_name="s")


@functools.partial(
    pl.kernel, mesh=_mesh,
    out_type=jax.ShapeDtypeStruct((NC, NP), jnp.float32),
    scratch_types=[
        pltpu.VMEM((K, CH), jnp.int32),
        pltpu.VMEM((CH,), jnp.float32),
        pltpu.VMEM_SHARED((NP,), jnp.float32),
    ],
)
def _sc_degree(dst_hbm, zeros_hbm, out_hbm, idx_v, ones_v, acc_sh):
    cid = lax.axis_index("c")
    sid = lax.axis_index("s")
    for i in range(CH // 16):
        ones_v[pl.ds(i * 16, 16)] = jnp.ones((16,), jnp.float32)
    pltpu.sync_copy(zeros_hbm.at[pl.ds(sid * RPT, RPT)],
                    acc_sh.at[pl.ds(sid * RPT, RPT)])
    pltpu.sync_copy(dst_hbm.at[cid, sid], idx_v)
    plsc.subcore_barrier()

    def body(j, carry):
        pltpu.sync_copy(ones_v, acc_sh.at[idx_v.at[j]], add=True)
        return carry

    lax.fori_loop(0, K, body, 0)
    plsc.subcore_barrier()
    pltpu.sync_copy(acc_sh.at[pl.ds(sid * RPT, RPT)],
                    out_hbm.at[cid, pl.ds(sid * RPT, RPT)])


def _make_sc_scatter(feat):
    @functools.partial(
        pl.kernel, mesh=_mesh,
        compiler_params=pltpu.CompilerParams(use_tc_tiling_on_sc=False),
        out_type=jax.ShapeDtypeStruct((NC, NP, feat), jnp.float32),
        scratch_types=[
            pltpu.VMEM((2, 2, CH), jnp.int32),
            pltpu.VMEM((2, CH, feat), jnp.float32),
            pltpu.VMEM_SHARED((NP, feat), jnp.float32),
            pltpu.SemaphoreType.DMA,
            pltpu.SemaphoreType.DMA,
        ],
    )
    def _sc_scatter(y_hbm, pair_hbm, out_hbm, idx, rows, acc_sh,
                    sem_i, sem_g):
        cid = lax.axis_index("c")
        sid = lax.axis_index("s")
        kc = jnp.where(cid == 0, K0, K1)
        pltpu.sync_copy(y_hbm.at[pl.ds(sid * RPT, RPT)],
                        acc_sh.at[pl.ds(sid * RPT, RPT)])
        plsc.subcore_barrier()

        pltpu.sync_copy(pair_hbm.at[cid, sid, 0], idx.at[0])
        pltpu.async_copy(y_hbm.at[idx.at[0, 0]], rows.at[0], sem_g)
        pltpu.async_copy(pair_hbm.at[cid, sid, 1], idx.at[1], sem_i)

        def step(j, cur, nxt):
            pltpu.make_async_copy(y_hbm.at[idx.at[cur, 0]], rows.at[cur],
                                  sem_g).wait()

            @pl.when(j + 1 < kc)
            def _pf():
                pltpu.make_async_copy(pair_hbm.at[cid, sid, j + 1],
                                      idx.at[nxt], sem_i).wait()
                pltpu.async_copy(y_hbm.at[idx.at[nxt, 0]], rows.at[nxt],
                                 sem_g)

            pltpu.sync_copy(rows.at[cur], acc_sh.at[idx.at[cur, 1]],
                            add=True)

            @pl.when(j + 2 < kc)
            def _pfi():
                pltpu.async_copy(pair_hbm.at[cid, sid, j + 2], idx.at[cur],
                                 sem_i)

        def body(jj, carry):
            step(2 * jj, 0, 1)

            @pl.when(2 * jj + 1 < kc)
            def _odd():
                step(2 * jj + 1, 1, 0)

            return carry

        lax.fori_loop(0, (kc + 1) // 2, body, 0)
        plsc.subcore_barrier()
        pltpu.sync_copy(acc_sh.at[pl.ds(sid * RPT, RPT)],
                        out_hbm.at[cid, pl.ds(sid * RPT, RPT)])

    return _sc_scatter


_sc_scatter_h = _make_sc_scatter(F_HID)
_sc_scatter_o = _make_sc_scatter(F_OUT)


def _tc1_body(x_ref, w_ref, deg_ref, y_ref):
    dis = lax.rsqrt(deg_ref[0, :] + deg_ref[1, :] + 1.0)
    xw = jnp.dot(x_ref[...], w_ref[...], preferred_element_type=jnp.float32)
    y_ref[...] = xw * dis[:, None]


def _tc1(xp, W1, deg):
    return pl.pallas_call(
        _tc1_body,
        grid=(NP // BR,),
        in_specs=[
            pl.BlockSpec((BR, F_IN), lambda i: (i, 0)),
            pl.BlockSpec((F_IN, F_HID), lambda i: (0, 0)),
            pl.BlockSpec((NC, BR), lambda i: (0, i)),
        ],
        out_specs=pl.BlockSpec((BR, F_HID), lambda i: (i, 0)),
        out_shape=jax.ShapeDtypeStruct((NP, F_HID), jnp.float32),
    )(xp, W1, deg)


def _tc2_body(a_ref, y1_ref, deg_ref, b1_ref, w2_ref, y2_ref):
    i = pl.program_id(0)
    dis = lax.rsqrt(deg_ref[0, :] + deg_ref[1, :] + 1.0)
    agg = a_ref[0] + a_ref[1] - y1_ref[...]
    h = jnp.maximum(agg * dis[:, None] + b1_ref[...][None, :], 0.0)
    row = i * BR + lax.broadcasted_iota(jnp.int32, (BR, 1), 0)
    h = jnp.where(row < N, h, 0.0)
    hw = jnp.dot(h, w2_ref[...], preferred_element_type=jnp.float32)
    y2_ref[...] = hw * dis[:, None]


def _tc2(a, y1, deg, b1, W2):
    return pl.pallas_call(
        _tc2_body,
        grid=(NP // BR,),
        in_specs=[
            pl.BlockSpec((NC, BR, F_HID), lambda i: (0, i, 0)),
            pl.BlockSpec((BR, F_HID), lambda i: (i, 0)),
            pl.BlockSpec((NC, BR), lambda i: (0, i)),
            pl.BlockSpec((F_HID,), lambda i: (0,)),
            pl.BlockSpec((F_HID, F_OUT), lambda i: (0, 0)),
        ],
        out_specs=pl.BlockSpec((BR, F_OUT), lambda i: (i, 0)),
        out_shape=jax.ShapeDtypeStruct((NP, F_OUT), jnp.float32),
    )(a, y1, deg, b1, W2)


def _tc3_body(c_ref, y2_ref, deg_ref, b2_ref, o_ref):
    dis = lax.rsqrt(deg_ref[0, :] + deg_ref[1, :] + 1.0)
    logits = (c_ref[0] + c_ref[1] - y2_ref[...]) * dis[:, None] \
        + b2_ref[...][None, :]
    m = jnp.max(logits, axis=1, keepdims=True)
    e = jnp.exp(logits - m)
    o_ref[...] = e / jnp.sum(e, axis=1, keepdims=True)


def _tc3(c, y2, deg, b2):
    return pl.pallas_call(
        _tc3_body,
        grid=(NP // BR,),
        in_specs=[
            pl.BlockSpec((NC, BR, F_OUT), lambda i: (0, i, 0)),
            pl.BlockSpec((BR, F_OUT), lambda i: (i, 0)),
            pl.BlockSpec((NC, BR), lambda i: (0, i)),
            pl.BlockSpec((F_OUT,), lambda i: (0,)),
        ],
        out_specs=pl.BlockSpec((BR, F_OUT), lambda i: (i, 0)),
        out_shape=jax.ShapeDtypeStruct((NP, F_OUT), jnp.float32),
    )(c, y2, deg, b2)


def kernel(x, edge_index, W1, b1, W2, b2):
    ei = edge_index.astype(jnp.int32)
    pad_s = jnp.full((EP - E,), N, jnp.int32)
    pad_d = N + jnp.arange(EP - E, dtype=jnp.int32) % (NP - N)
    flat_s = jnp.concatenate([ei[0], pad_s])
    flat_d = jnp.concatenate([ei[1], pad_d])
    n0 = NS * K0 * CH
    pair = jnp.full((NC, NS, KMAX, 2, CH), N, jnp.int32)
    pair = pair.at[0, :, :K0].set(jnp.stack(
        [flat_s[:n0].reshape(NS, K0, CH), flat_d[:n0].reshape(NS, K0, CH)],
        axis=2))
    pair = pair.at[1, :, :K1].set(jnp.stack(
        [flat_s[n0:].reshape(NS, K1, CH), flat_d[n0:].reshape(NS, K1, CH)],
        axis=2))
    xp = jnp.pad(x, ((0, NP - N), (0, 0)))
    zeros_np = jnp.zeros((NP,), jnp.float32)

    deg = _sc_degree(flat_d.reshape(NC, NS, K, CH), zeros_np)
    y1 = _tc1(xp, W1, deg)
    a = _sc_scatter_h(y1, pair)
    y2 = _tc2(a, y1, deg, b1, W2)
    c = _sc_scatter_o(y2, pair)
    out = _tc3(c, y2, deg, b2)
    return out[:N]


# speedup vs baseline: 1.0636x; 1.0636x over previous
import functools

import jax
import jax.numpy as jnp
from jax import lax
from jax.experimental import pallas as pl
from jax.experimental.pallas import tpu as pltpu
from jax.experimental.pallas import tpu_sc as plsc

N = 10000
NP = 10240
F_IN = 128
F_HID = 128
F_OUT = 64
E = 320000
NC = 2
NS = 16
CH = 128
K = -(-E // (NC * NS * CH))
EP = NC * NS * CH * K
K0 = 107
K1 = 2 * K - K0
KMAX = max(K0, K1)
RPT = NP // NS
BR = NP // 8

_mesh = plsc.VectorSubcoreMesh(core_axis_name="c", subcore_axis_name="s")


@functools.partial(
    pl.kernel, mesh=_mesh,
    out_type=jax.ShapeDtypeStruct((NC, NP), jnp.float32),
    scratch_types=[
        pltpu.VMEM((K, CH), jnp.int32),
        pltpu.VMEM((CH,), jnp.float32),
        pltpu.VMEM_SHARED((NP,), jnp.float32),
    ],
)
def _sc_degree(dst_hbm, zeros_hbm, out_hbm, idx_v, ones_v, acc_sh):
    cid = lax.axis_index("c")
    sid = lax.axis_index("s")
    for i in range(CH // 16):
        ones_v[pl.ds(i * 16, 16)] = jnp.ones((16,), jnp.float32)
    pltpu.sync_copy(zeros_hbm.at[pl.ds(sid * RPT, RPT)],
                    acc_sh.at[pl.ds(sid * RPT, RPT)])
    pltpu.sync_copy(dst_hbm.at[cid, sid], idx_v)
    plsc.subcore_barrier()

    def body(j, carry):
        pltpu.sync_copy(ones_v, acc_sh.at[idx_v.at[j]], add=True)
        return carry

    lax.fori_loop(0, K, body, 0)
    plsc.subcore_barrier()
    pltpu.sync_copy(acc_sh.at[pl.ds(sid * RPT, RPT)],
                    out_hbm.at[cid, pl.ds(sid * RPT, RPT)])


def _make_sc_scatter(feat):
    @functools.partial(
        pl.kernel, mesh=_mesh,
        compiler_params=pltpu.CompilerParams(use_tc_tiling_on_sc=False),
        out_type=jax.ShapeDtypeStruct((NC, NP, feat), jnp.float32),
        scratch_types=[
            pltpu.VMEM((2, 2, CH), jnp.int32),
            pltpu.VMEM((2, CH, feat), jnp.float32),
            pltpu.VMEM_SHARED((NP, feat), jnp.float32),
            pltpu.SemaphoreType.DMA,
            pltpu.SemaphoreType.DMA,
        ],
    )
    def _sc_scatter(y_hbm, pair_hbm, out_hbm, idx, rows, acc_sh,
                    sem_i, sem_g):
        cid = lax.axis_index("c")
        sid = lax.axis_index("s")
        kc = jnp.where(cid == 0, K0, K1)
        pltpu.sync_copy(y_hbm.at[pl.ds(sid * RPT, RPT)],
                        acc_sh.at[pl.ds(sid * RPT, RPT)])
        plsc.subcore_barrier()

        pltpu.sync_copy(pair_hbm.at[cid, sid, 0], idx.at[0])
        pltpu.async_copy(y_hbm.at[idx.at[0, 0]], rows.at[0], sem_g)
        pltpu.async_copy(pair_hbm.at[cid, sid, 1], idx.at[1], sem_i)

        def step(j, cur, nxt):
            pltpu.make_async_copy(y_hbm.at[idx.at[cur, 0]], rows.at[cur],
                                  sem_g).wait()

            @pl.when(j + 1 < kc)
            def _pf():
                pltpu.make_async_copy(pair_hbm.at[cid, sid, j + 1],
                                      idx.at[nxt], sem_i).wait()
                pltpu.async_copy(y_hbm.at[idx.at[nxt, 0]], rows.at[nxt],
                                 sem_g)

            pltpu.sync_copy(rows.at[cur], acc_sh.at[idx.at[cur, 1]],
                            add=True)

            @pl.when(j + 2 < kc)
            def _pfi():
                pltpu.async_copy(pair_hbm.at[cid, sid, j + 2], idx.at[cur],
                                 sem_i)

        def body(jj, carry):
            step(2 * jj, 0, 1)

            @pl.when(2 * jj + 1 < kc)
            def _odd():
                step(2 * jj + 1, 1, 0)

            return carry

        lax.fori_loop(0, (kc + 1) // 2, body, 0)
        plsc.subcore_barrier()
        pltpu.sync_copy(acc_sh.at[pl.ds(sid * RPT, RPT)],
                        out_hbm.at[cid, pl.ds(sid * RPT, RPT)])

    return _sc_scatter


_sc_scatter_h = _make_sc_scatter(F_HID)
_sc_scatter_o = _make_sc_scatter(F_OUT)


def _tc1_body(x_ref, w_ref, deg_ref, y_ref):
    dis = lax.rsqrt(deg_ref[0, :] + deg_ref[1, :] + 1.0)
    xw = jnp.dot(x_ref[...], w_ref[...], preferred_element_type=jnp.float32)
    y_ref[...] = xw * dis[:, None]


def _tc1(xp, W1, deg):
    return pl.pallas_call(
        _tc1_body,
        grid=(NP // BR,),
        in_specs=[
            pl.BlockSpec((BR, F_IN), lambda i: (i, 0)),
            pl.BlockSpec((F_IN, F_HID), lambda i: (0, 0)),
            pl.BlockSpec((NC, BR), lambda i: (0, i)),
        ],
        out_specs=pl.BlockSpec((BR, F_HID), lambda i: (i, 0)),
        out_shape=jax.ShapeDtypeStruct((NP, F_HID), jnp.float32),
    )(xp, W1, deg)


def _tc2_body(a_ref, y1_ref, deg_ref, b1_ref, w2_ref, y2_ref):
    i = pl.program_id(0)
    dis = lax.rsqrt(deg_ref[0, :] + deg_ref[1, :] + 1.0)
    agg = a_ref[0] + a_ref[1] - y1_ref[...]
    h = jnp.maximum(agg * dis[:, None] + b1_ref[...][None, :], 0.0)
    row = i * BR + lax.broadcasted_iota(jnp.int32, (BR, 1), 0)
    h = jnp.where(row < N, h, 0.0)
    hw = jnp.dot(h, w2_ref[...], preferred_element_type=jnp.float32)
    y2_ref[...] = hw * dis[:, None]


def _tc2(a, y1, deg, b1, W2):
    return pl.pallas_call(
        _tc2_body,
        grid=(NP // BR,),
        in_specs=[
            pl.BlockSpec((NC, BR, F_HID), lambda i: (0, i, 0)),
            pl.BlockSpec((BR, F_HID), lambda i: (i, 0)),
            pl.BlockSpec((NC, BR), lambda i: (0, i)),
            pl.BlockSpec((F_HID,), lambda i: (0,)),
            pl.BlockSpec((F_HID, F_OUT), lambda i: (0, 0)),
        ],
        out_specs=pl.BlockSpec((BR, F_OUT), lambda i: (i, 0)),
        out_shape=jax.ShapeDtypeStruct((NP, F_OUT), jnp.float32),
    )(a, y1, deg, b1, W2)


def _tc3_body(c_ref, y2_ref, deg_ref, b2_ref, o_ref):
    dis = lax.rsqrt(deg_ref[0, :] + deg_ref[1, :] + 1.0)
    logits = (c_ref[0] + c_ref[1] - y2_ref[...]) * dis[:, None] \
        + b2_ref[...][None, :]
    m = jnp.max(logits, axis=1, keepdims=True)
    e = jnp.exp(logits - m)
    o_ref[...] = e / jnp.sum(e, axis=1, keepdims=True)


def _tc3(c, y2, deg, b2):
    return pl.pallas_call(
        _tc3_body,
        grid=(NP // BR,),
        in_specs=[
            pl.BlockSpec((NC, BR, F_OUT), lambda i: (0, i, 0)),
            pl.BlockSpec((BR, F_OUT), lambda i: (i, 0)),
            pl.BlockSpec((NC, BR), lambda i: (0, i)),
            pl.BlockSpec((F_OUT,), lambda i: (0,)),
        ],
        out_specs=pl.BlockSpec((BR, F_OUT), lambda i: (i, 0)),
        out_shape=jax.ShapeDtypeStruct((NP, F_OUT), jnp.float32),
    )(c, y2, deg, b2)


def kernel(x, edge_index, W1, b1, W2, b2):
    ei = edge_index.astype(jnp.int32)
    pad_s = jnp.full((EP - E,), N, jnp.int32)
    pad_d = N + jnp.arange(EP - E, dtype=jnp.int32) % (NP - N)
    flat_s = jnp.concatenate([ei[0], pad_s])
    flat_d = jnp.concatenate([ei[1], pad_d])
    n0 = NS * K0 * CH
    pair = jnp.full((NC, NS, KMAX, 2, CH), N, jnp.int32)
    pair = pair.at[0, :, :K0].set(jnp.stack(
        [flat_s[:n0].reshape(NS, K0, CH), flat_d[:n0].reshape(NS, K0, CH)],
        axis=2))
    pair = pair.at[1, :, :K1].set(jnp.stack(
        [flat_s[n0:].reshape(NS, K1, CH), flat_d[n0:].reshape(NS, K1, CH)],
        axis=2))
    xp = jnp.pad(x, ((0, NP - N), (0, 0)))
    zeros_np = jnp.zeros((NP,), jnp.float32)

    deg = _sc_degree(flat_d.reshape(NC, NS, K, CH), zeros_np)
    y1 = _tc1(xp, W1, deg)
    a = _sc_scatter_h(y1, pair)
    y2 = _tc2(a, y1, deg, b1, W2)
    c = _sc_scatter_o(y2, pair)
    out = _tc3(c, y2, deg, b2)
    return out[:N]


# speedup vs baseline: 1.2644x; 1.1888x over previous
import functools

import jax
import jax.numpy as jnp
from jax import lax
from jax.experimental import pallas as pl
from jax.experimental.pallas import tpu as pltpu
from jax.experimental.pallas import tpu_sc as plsc

N = 10000
NP = 10240
F_IN = 128
F_HID = 128
F_OUT = 64
E = 320000
NC = 2
NS = 16
CH = 128
KD = -(-E // (NC * NS * CH))
EPD = NC * NS * CH * KD
K2 = -(-E // (NS * CH))
EP2 = NS * CH * K2
RPT = NP // NS
BR = NP // 8

_mesh = plsc.VectorSubcoreMesh(core_axis_name="c", subcore_axis_name="s")


@functools.partial(
    pl.kernel, mesh=_mesh,
    out_type=jax.ShapeDtypeStruct((NC, NP), jnp.float32),
    scratch_types=[
        pltpu.VMEM((KD, CH), jnp.int32),
        pltpu.VMEM((CH,), jnp.float32),
        pltpu.VMEM_SHARED((NP,), jnp.float32),
    ],
)
def _sc_degree(dst_hbm, zeros_hbm, out_hbm, idx_v, ones_v, acc_sh):
    cid = lax.axis_index("c")
    sid = lax.axis_index("s")
    for i in range(CH // 16):
        ones_v[pl.ds(i * 16, 16)] = jnp.ones((16,), jnp.float32)
    pltpu.sync_copy(zeros_hbm.at[pl.ds(sid * RPT, RPT)],
                    acc_sh.at[pl.ds(sid * RPT, RPT)])
    pltpu.sync_copy(dst_hbm.at[cid, sid], idx_v)
    plsc.subcore_barrier()

    def body(j, carry):
        pltpu.sync_copy(ones_v, acc_sh.at[idx_v.at[j]], add=True)
        return carry

    lax.fori_loop(0, KD, body, 0)
    plsc.subcore_barrier()
    pltpu.sync_copy(acc_sh.at[pl.ds(sid * RPT, RPT)],
                    out_hbm.at[cid, pl.ds(sid * RPT, RPT)])


def _make_sc_scatter(f2):
    @functools.partial(
        pl.kernel, mesh=_mesh,
        compiler_params=pltpu.CompilerParams(use_tc_tiling_on_sc=False),
        out_type=jax.ShapeDtypeStruct((NC, NP, f2), jnp.float32),
        scratch_types=[
            pltpu.VMEM((2, 2, CH), jnp.int32),
            pltpu.VMEM((2, CH, f2), jnp.float32),
            pltpu.VMEM_SHARED((NP, f2), jnp.float32),
            pltpu.VMEM_SHARED((NP, f2), jnp.float32),
            pltpu.SemaphoreType.DMA,
            pltpu.SemaphoreType.DMA,
        ],
    )
    def _sc_scatter(y_hbm, pair_hbm, out_hbm, idx, rows, y_sh, acc_sh,
                    sem_i, sem_g):
        cid = lax.axis_index("c")
        sid = lax.axis_index("s")
        sl = pl.ds(sid * RPT, RPT)
        pltpu.sync_copy(y_hbm.at[cid, sl], y_sh.at[sl])
        pltpu.sync_copy(y_hbm.at[cid, sl], acc_sh.at[sl])
        pltpu.sync_copy(pair_hbm.at[sid, 0], idx.at[0])
        plsc.subcore_barrier()

        pltpu.async_copy(y_sh.at[idx.at[0, 0]], rows.at[0], sem_g)
        pltpu.async_copy(pair_hbm.at[sid, 1], idx.at[1], sem_i)

        def step(j, cur, nxt):
            pltpu.make_async_copy(y_sh.at[idx.at[cur, 0]], rows.at[cur],
                                  sem_g).wait()

            @pl.when(j + 1 < K2)
            def _pf():
                pltpu.make_async_copy(pair_hbm.at[sid, j + 1],
                                      idx.at[nxt], sem_i).wait()
                pltpu.async_copy(y_sh.at[idx.at[nxt, 0]], rows.at[nxt],
                                 sem_g)

            pltpu.sync_copy(rows.at[cur], acc_sh.at[idx.at[cur, 1]],
                            add=True)

            @pl.when(j + 2 < K2)
            def _pfi():
                pltpu.async_copy(pair_hbm.at[sid, j + 2], idx.at[cur],
                                 sem_i)

        def body(jj, carry):
            step(2 * jj, 0, 1)

            @pl.when(2 * jj + 1 < K2)
            def _odd():
                step(2 * jj + 1, 1, 0)

            return carry

        lax.fori_loop(0, (K2 + 1) // 2, body, 0)
        plsc.subcore_barrier()
        pltpu.sync_copy(acc_sh.at[sl], out_hbm.at[cid, sl])

    return _sc_scatter


_sc_scatter_h = _make_sc_scatter(F_HID // 2)
_sc_scatter_o = _make_sc_scatter(F_OUT // 2)


def _tc1_body(x_ref, w_ref, deg_ref, y_ref):
    dis = lax.rsqrt(deg_ref[0, :] + deg_ref[1, :] + 1.0)
    xw = jnp.dot(x_ref[...], w_ref[...], preferred_element_type=jnp.float32)
    y = xw * dis[:, None]
    y_ref[0] = y[:, :F_HID // 2]
    y_ref[1] = y[:, F_HID // 2:]


def _tc1(xp, W1, deg):
    return pl.pallas_call(
        _tc1_body,
        grid=(NP // BR,),
        in_specs=[
            pl.BlockSpec((BR, F_IN), lambda i: (i, 0)),
            pl.BlockSpec((F_IN, F_HID), lambda i: (0, 0)),
            pl.BlockSpec((NC, BR), lambda i: (0, i)),
        ],
        out_specs=pl.BlockSpec((NC, BR, F_HID // 2), lambda i: (0, i, 0)),
        out_shape=jax.ShapeDtypeStruct((NC, NP, F_HID // 2), jnp.float32),
    )(xp, W1, deg)


def _tc2_body(a_ref, deg_ref, b1_ref, w2_ref, y2_ref):
    i = pl.program_id(0)
    dis = lax.rsqrt(deg_ref[0, :] + deg_ref[1, :] + 1.0)
    agg = jnp.concatenate([a_ref[0], a_ref[1]], axis=1)
    h = jnp.maximum(agg * dis[:, None] + b1_ref[...][None, :], 0.0)
    row = i * BR + lax.broadcasted_iota(jnp.int32, (BR, 1), 0)
    h = jnp.where(row < N, h, 0.0)
    hw = jnp.dot(h, w2_ref[...], preferred_element_type=jnp.float32)
    y2 = hw * dis[:, None]
    y2_ref[0] = y2[:, :F_OUT // 2]
    y2_ref[1] = y2[:, F_OUT // 2:]


def _tc2(a, deg, b1, W2):
    return pl.pallas_call(
        _tc2_body,
        grid=(NP // BR,),
        in_specs=[
            pl.BlockSpec((NC, BR, F_HID // 2), lambda i: (0, i, 0)),
            pl.BlockSpec((NC, BR), lambda i: (0, i)),
            pl.BlockSpec((F_HID,), lambda i: (0,)),
            pl.BlockSpec((F_HID, F_OUT), lambda i: (0, 0)),
        ],
        out_specs=pl.BlockSpec((NC, BR, F_OUT // 2), lambda i: (0, i, 0)),
        out_shape=jax.ShapeDtypeStruct((NC, NP, F_OUT // 2), jnp.float32),
    )(a, deg, b1, W2)


def _tc3_body(c_ref, deg_ref, b2_ref, o_ref):
    dis = lax.rsqrt(deg_ref[0, :] + deg_ref[1, :] + 1.0)
    agg = jnp.concatenate([c_ref[0], c_ref[1]], axis=1)
    logits = agg * dis[:, None] + b2_ref[...][None, :]
    m = jnp.max(logits, axis=1, keepdims=True)
    e = jnp.exp(logits - m)
    o_ref[...] = e / jnp.sum(e, axis=1, keepdims=True)


def _tc3(c, deg, b2):
    return pl.pallas_call(
        _tc3_body,
        grid=(NP // BR,),
        in_specs=[
            pl.BlockSpec((NC, BR, F_OUT // 2), lambda i: (0, i, 0)),
            pl.BlockSpec((NC, BR), lambda i: (0, i)),
            pl.BlockSpec((F_OUT,), lambda i: (0,)),
        ],
        out_specs=pl.BlockSpec((BR, F_OUT), lambda i: (i, 0)),
        out_shape=jax.ShapeDtypeStruct((NP, F_OUT), jnp.float32),
    )(c, deg, b2)


def kernel(x, edge_index, W1, b1, W2, b2):
    ei = edge_index.astype(jnp.int32)
    pad_s2 = jnp.full((EP2 - E,), N, jnp.int32)
    pad_d2 = N + jnp.arange(EP2 - E, dtype=jnp.int32) % (NP - N)
    pair = jnp.stack(
        [jnp.concatenate([ei[0], pad_s2]).reshape(NS, K2, CH),
         jnp.concatenate([ei[1], pad_d2]).reshape(NS, K2, CH)],
        axis=2)
    pad_dd = N + jnp.arange(EPD - E, dtype=jnp.int32) % (NP - N)
    dst_deg = jnp.concatenate([ei[1], pad_dd]).reshape(NC, NS, KD, CH)
    xp = jnp.pad(x, ((0, NP - N), (0, 0)))
    zeros_np = jnp.zeros((NP,), jnp.float32)

    deg = _sc_degree(dst_deg, zeros_np)
    y1 = _tc1(xp, W1, deg)
    a = _sc_scatter_h(y1, pair)
    y2 = _tc2(a, deg, b1, W2)
    c = _sc_scatter_o(y2, pair)
    out = _tc3(c, deg, b2)
    return out[:N]


# speedup vs baseline: 1.7008x; 1.3451x over previous
import functools

import jax
import jax.numpy as jnp
from jax import lax
from jax.experimental import pallas as pl
from jax.experimental.pallas import tpu as pltpu
from jax.experimental.pallas import tpu_sc as plsc

N = 10000
NP = 10240
F_IN = 128
F_HID = 128
F_OUT = 64
E = 320000
NC = 2
NS = 16
CH = 128
KD = -(-E // (NC * NS * CH))
EPD = NC * NS * CH * KD
K2 = -(-E // (NS * CH))
EP2 = NS * CH * K2
RPT = NP // NS
BR = NP // 8

_mesh = plsc.VectorSubcoreMesh(core_axis_name="c", subcore_axis_name="s")


@functools.partial(
    pl.kernel, mesh=_mesh,
    out_type=jax.ShapeDtypeStruct((NC, NP), jnp.float32),
    scratch_types=[
        pltpu.VMEM((KD, CH), jnp.int32),
        pltpu.VMEM((CH,), jnp.float32),
        pltpu.VMEM_SHARED((NP,), jnp.float32),
    ],
)
def _sc_degree(dst_hbm, zeros_hbm, out_hbm, idx_v, ones_v, acc_sh):
    cid = lax.axis_index("c")
    sid = lax.axis_index("s")
    for i in range(CH // 16):
        ones_v[pl.ds(i * 16, 16)] = jnp.ones((16,), jnp.float32)
    pltpu.sync_copy(zeros_hbm.at[pl.ds(sid * RPT, RPT)],
                    acc_sh.at[pl.ds(sid * RPT, RPT)])
    pltpu.sync_copy(dst_hbm.at[cid, sid], idx_v)
    plsc.subcore_barrier()

    def body(j, carry):
        pltpu.sync_copy(ones_v, acc_sh.at[idx_v.at[j]], add=True)
        return carry

    lax.fori_loop(0, KD, body, 0)
    plsc.subcore_barrier()
    pltpu.sync_copy(acc_sh.at[pl.ds(sid * RPT, RPT)],
                    out_hbm.at[cid, pl.ds(sid * RPT, RPT)])


def _make_sc_scatter(f2):
    @functools.partial(
        pl.kernel, mesh=_mesh,
        compiler_params=pltpu.CompilerParams(use_tc_tiling_on_sc=False),
        out_type=jax.ShapeDtypeStruct((NC, NP, f2), jnp.float32),
        scratch_types=[
            pltpu.VMEM((4, 2, CH), jnp.int32),
            pltpu.VMEM((3, CH, f2), jnp.float32),
            pltpu.VMEM_SHARED((NP, f2), jnp.float32),
            pltpu.VMEM_SHARED((NP, f2), jnp.float32),
            pltpu.SemaphoreType.DMA,
            pltpu.SemaphoreType.DMA,
            pltpu.SemaphoreType.DMA,
        ],
    )
    def _sc_scatter(src_hbm, dst_hbm, y_hbm, out_hbm, idx, rows, y_sh,
                    acc_sh, sem_i, sem_g, sem_s):
        cid = lax.axis_index("c")
        sid = lax.axis_index("s")
        sl = pl.ds(sid * RPT, RPT)
        pltpu.sync_copy(y_hbm.at[cid, sl], y_sh.at[sl])
        pltpu.sync_copy(y_hbm.at[cid, sl], acc_sh.at[sl])
        pltpu.sync_copy(src_hbm.at[sid, 0], idx.at[0, 0])
        pltpu.sync_copy(dst_hbm.at[sid, 0], idx.at[0, 1])
        plsc.subcore_barrier()

        pltpu.async_copy(y_sh.at[idx.at[0, 0]], rows.at[0], sem_g)
        pltpu.async_copy(src_hbm.at[sid, 1], idx.at[1, 0], sem_i)
        pltpu.async_copy(dst_hbm.at[sid, 1], idx.at[1, 1], sem_i)

        def body(j, carry):
            rs = lax.rem(j, 3)
            ds = lax.rem(j, 4)
            pltpu.make_async_copy(y_sh.at[idx.at[ds, 0]], rows.at[rs],
                                  sem_g).wait()
            cp = pltpu.make_async_copy(rows.at[rs], acc_sh.at[idx.at[ds, 1]],
                                       sem_s)
            cp.start(add=True)

            @pl.when(j >= 2)
            def _drain():
                pltpu.make_async_copy(rows.at[lax.rem(j + 1, 3)],
                                      acc_sh.at[idx.at[lax.rem(j + 2, 4), 1]],
                                      sem_s).wait()

            @pl.when(j + 1 < K2)
            def _pf():
                ns = lax.rem(j + 1, 4)
                pltpu.make_async_copy(src_hbm.at[sid, j + 1], idx.at[ns, 0],
                                      sem_i).wait()
                pltpu.make_async_copy(dst_hbm.at[sid, j + 1], idx.at[ns, 1],
                                      sem_i).wait()
                pltpu.async_copy(y_sh.at[idx.at[ns, 0]], rows.at[lax.rem(
                    j + 1, 3)], sem_g)

            @pl.when(j + 2 < K2)
            def _pfi():
                ns2 = lax.rem(j + 2, 4)
                pltpu.async_copy(src_hbm.at[sid, j + 2], idx.at[ns2, 0],
                                 sem_i)
                pltpu.async_copy(dst_hbm.at[sid, j + 2], idx.at[ns2, 1],
                                 sem_i)

            return carry

        lax.fori_loop(0, K2, body, 0)
        pltpu.make_async_copy(rows.at[lax.rem(K2 - 2, 3)],
                              acc_sh.at[idx.at[lax.rem(K2 - 2, 4), 1]],
                              sem_s).wait()
        pltpu.make_async_copy(rows.at[lax.rem(K2 - 1, 3)],
                              acc_sh.at[idx.at[lax.rem(K2 - 1, 4), 1]],
                              sem_s).wait()
        plsc.subcore_barrier()
        pltpu.sync_copy(acc_sh.at[sl], out_hbm.at[cid, sl])

    return _sc_scatter


_sc_scatter_h = _make_sc_scatter(F_HID // 2)
_sc_scatter_o = _make_sc_scatter(F_OUT // 2)


def _tc1_body(x_ref, w_ref, deg_ref, y_ref):
    dis = lax.rsqrt(deg_ref[0, :] + deg_ref[1, :] + 1.0)
    xw = jnp.dot(x_ref[...], w_ref[...], preferred_element_type=jnp.float32)
    y = xw * dis[:, None]
    y_ref[0] = y[:, :F_HID // 2]
    y_ref[1] = y[:, F_HID // 2:]


def _tc1(xp, W1, deg):
    return pl.pallas_call(
        _tc1_body,
        grid=(NP // BR,),
        in_specs=[
            pl.BlockSpec((BR, F_IN), lambda i: (i, 0)),
            pl.BlockSpec((F_IN, F_HID), lambda i: (0, 0)),
            pl.BlockSpec((NC, BR), lambda i: (0, i)),
        ],
        out_specs=pl.BlockSpec((NC, BR, F_HID // 2), lambda i: (0, i, 0)),
        out_shape=jax.ShapeDtypeStruct((NC, NP, F_HID // 2), jnp.float32),
    )(xp, W1, deg)


def _tc2_body(a_ref, deg_ref, b1_ref, w2_ref, y2_ref):
    i = pl.program_id(0)
    dis = lax.rsqrt(deg_ref[0, :] + deg_ref[1, :] + 1.0)
    agg = jnp.concatenate([a_ref[0], a_ref[1]], axis=1)
    h = jnp.maximum(agg * dis[:, None] + b1_ref[...][None, :], 0.0)
    row = i * BR + lax.broadcasted_iota(jnp.int32, (BR, 1), 0)
    h = jnp.where(row < N, h, 0.0)
    hw = jnp.dot(h, w2_ref[...], preferred_element_type=jnp.float32)
    y2 = hw * dis[:, None]
    y2_ref[0] = y2[:, :F_OUT // 2]
    y2_ref[1] = y2[:, F_OUT // 2:]


def _tc2(a, deg, b1, W2):
    return pl.pallas_call(
        _tc2_body,
        grid=(NP // BR,),
        in_specs=[
            pl.BlockSpec((NC, BR, F_HID // 2), lambda i: (0, i, 0)),
            pl.BlockSpec((NC, BR), lambda i: (0, i)),
            pl.BlockSpec((F_HID,), lambda i: (0,)),
            pl.BlockSpec((F_HID, F_OUT), lambda i: (0, 0)),
        ],
        out_specs=pl.BlockSpec((NC, BR, F_OUT // 2), lambda i: (0, i, 0)),
        out_shape=jax.ShapeDtypeStruct((NC, NP, F_OUT // 2), jnp.float32),
    )(a, deg, b1, W2)


def _tc3_body(c_ref, deg_ref, b2_ref, o_ref):
    dis = lax.rsqrt(deg_ref[0, :] + deg_ref[1, :] + 1.0)
    agg = jnp.concatenate([c_ref[0], c_ref[1]], axis=1)
    logits = agg * dis[:, None] + b2_ref[...][None, :]
    m = jnp.max(logits, axis=1, keepdims=True)
    e = jnp.exp(logits - m)
    o_ref[...] = e / jnp.sum(e, axis=1, keepdims=True)


def _tc3(c, deg, b2):
    return pl.pallas_call(
        _tc3_body,
        grid=(NP // BR,),
        in_specs=[
            pl.BlockSpec((NC, BR, F_OUT // 2), lambda i: (0, i, 0)),
            pl.BlockSpec((NC, BR), lambda i: (0, i)),
            pl.BlockSpec((F_OUT,), lambda i: (0,)),
        ],
        out_specs=pl.BlockSpec((BR, F_OUT), lambda i: (i, 0)),
        out_shape=jax.ShapeDtypeStruct((NP, F_OUT), jnp.float32),
    )(c, deg, b2)


def kernel(x, edge_index, W1, b1, W2, b2):
    ei = edge_index.astype(jnp.int32)
    pad_s2 = jnp.full((EP2 - E,), N, jnp.int32)
    pad_d2 = N + jnp.arange(EP2 - E, dtype=jnp.int32) % (NP - N)
    src2 = jnp.concatenate([ei[0], pad_s2]).reshape(NS, K2, CH)
    dst2 = jnp.concatenate([ei[1], pad_d2]).reshape(NS, K2, CH)
    pad_dd = N + jnp.arange(EPD - E, dtype=jnp.int32) % (NP - N)
    dst_deg = jnp.concatenate([ei[1], pad_dd]).reshape(NC, NS, KD, CH)
    xp = jnp.pad(x, ((0, NP - N), (0, 0)))
    zeros_np = jnp.zeros((NP,), jnp.float32)

    deg = _sc_degree(dst_deg, zeros_np)
    y1 = _tc1(xp, W1, deg)
    a = _sc_scatter_h(src2, dst2, y1)
    y2 = _tc2(a, deg, b1, W2)
    c = _sc_scatter_o(src2, dst2, y2)
    out = _tc3(c, deg, b2)
    return out[:N]
